# baseline (speedup 1.0000x reference)
import functools
import math

import jax
import jax.numpy as jnp
from jax import lax
from jax.experimental import pallas as pl
from jax.experimental.pallas import tpu as pltpu
from jax.experimental.pallas import tpu_sc as plsc

D_MODEL = 64
D_PAD = 128
SCALE = math.sqrt(D_MODEL)

NUM_CORES = 2
NUM_SUBCORES = 16
NUM_WORKERS = NUM_CORES * NUM_SUBCORES
LANES = 16
NBUF = 4
LOOKAHEAD = 2

VOCAB = 1000000
MAIN_TILES = VOCAB // D_PAD
MAIN_ROWS = MAIN_TILES * D_PAD
TAIL_ROWS = VOCAB - MAIN_ROWS
OUT_STRIDE = D_MODEL + 1
TBUF = 4


@jax.jit
def _transpose_scale(t4, tail_p):
    mesh = plsc.VectorSubcoreMesh(core_axis_name="c", subcore_axis_name="s")
    slabs_per_worker = MAIN_TILES // NUM_WORKERS
    extra = MAIN_TILES - slabs_per_worker * NUM_WORKERS

    @functools.partial(
        pl.kernel,
        mesh=mesh,
        out_type=jax.ShapeDtypeStruct((VOCAB, D_MODEL), jnp.float32),
        scratch_types=[
            pltpu.VMEM((TBUF, 8, 8, D_PAD), jnp.float32),
            pltpu.VMEM((TBUF, D_PAD, OUT_STRIDE), jnp.float32),
            pltpu.VMEM((TAIL_ROWS, D_MODEL), jnp.float32),
            [pltpu.SemaphoreType.DMA] * TBUF,
            [pltpu.SemaphoreType.DMA] * TBUF,
        ],
        compiler_params=pltpu.CompilerParams(
            use_tc_tiling_on_sc=False, needs_layout_passes=False
        ),
    )
    def ka(t4_hbm, tail_hbm, outp_hbm, inb, outb, tailb, gsems, ssems):
        wid = lax.axis_index("s") * NUM_CORES + lax.axis_index("c")
        nslab = slabs_per_worker + jnp.where(wid < extra, 1, 0)

        @pl.when(wid == 0)
        def _():
            pltpu.sync_copy(tail_hbm, tailb)

            @plsc.parallel_loop(0, TAIL_ROWS, unroll=4)
            def _(r):
                for jj in range(D_MODEL // LANES):
                    sl = pl.ds(jj * LANES, LANES)
                    tailb[r, sl] = tailb[r, sl] * SCALE

            pltpu.sync_copy(tailb, outp_hbm.at[pl.ds(MAIN_ROWS, TAIL_ROWS)])

        def slab_of(k):
            return wid + k * NUM_WORKERS

        def gather(k, b, sem):
            src = t4_hbm.at[:, slab_of(k)]
            return pltpu.make_async_copy(src, inb.at[b], sem)

        def scatter(k, b, sem):
            dst = outp_hbm.at[pl.ds(slab_of(k) * D_PAD, D_PAD)]
            src = outb.at[b, :, pl.ds(0, D_MODEL)]
            return pltpu.make_async_copy(src, dst, sem)

        gather(0, 0, gsems[0]).start()

        def body2(t, _):
            for b in range(TBUF):
                k = t * TBUF + b

                @pl.when(k < nslab)
                def _():
                    bn = (b + 1) % TBUF

                    @pl.when(k + 1 < nslab)
                    def _():
                        @pl.when(k + 1 >= TBUF)
                        def _():
                            scatter(k + 1 - TBUF, bn, ssems[bn]).wait()

                        gather(k + 1, bn, gsems[bn]).start()

                    gather(k, b, gsems[b]).wait()

                    @plsc.parallel_loop(0, D_PAD // LANES, unroll=2)
                    def _(v):
                        j0 = v * LANES
                        jvec = j0 + lax.iota(jnp.int32, LANES)
                        for d in range(D_MODEL):
                            val = inb[b, d // 8, d % 8, pl.ds(j0, LANES)] * SCALE
                            plsc.store_scatter(
                                outb.at[b],
                                [jvec, jnp.full((LANES,), d, jnp.int32)],
                                val,
                            )

                    scatter(k, b, ssems[b]).start()
            return 0

        max_outer = (slabs_per_worker + 1 + TBUF - 1) // TBUF
        lax.fori_loop(0, max_outer, body2, 0)

        for b in range(TBUF):
            @pl.when(nslab > b)
            def _(b=b):
                scatter(0, b, ssems[b]).wait()

    return ka(t4, tail_p)


@functools.partial(jax.jit, static_argnames=("total_rows", "chunk"))
def _embed(x_flat, table_p, *, total_rows, chunk):
    rows_per_worker = total_rows // NUM_WORKERS
    num_chunks = rows_per_worker // chunk
    assert num_chunks % NBUF == 0
    outer = num_chunks // NBUF
    mesh = plsc.VectorSubcoreMesh(core_axis_name="c", subcore_axis_name="s")

    @functools.partial(
        pl.kernel,
        mesh=mesh,
        out_type=jax.ShapeDtypeStruct((total_rows, D_PAD), jnp.float32),
        scratch_types=[
            pltpu.VMEM((rows_per_worker,), jnp.int32),
            pltpu.VMEM((NBUF, chunk, D_MODEL), jnp.float32),
            [pltpu.SemaphoreType.DMA] * NBUF,
            [pltpu.SemaphoreType.DMA] * NBUF,
        ],
        compiler_params=pltpu.CompilerParams(use_tc_tiling_on_sc=False),
    )
    def k(table_hbm, idx_hbm, out_hbm, idx_v, rows_v, gsems, ssems):
        wid = lax.axis_index("s") * NUM_CORES + lax.axis_index("c")
        base = wid * rows_per_worker
        pltpu.sync_copy(idx_hbm.at[pl.ds(base, rows_per_worker)], idx_v)

        def gather(i, b, sem):
            idx_sl = idx_v.at[pl.ds(i * chunk, chunk)]
            return pltpu.make_async_copy(table_hbm.at[idx_sl], rows_v.at[b], sem)

        def scatter(i, b, sem):
            dst = out_hbm.at[pl.ds(base + i * chunk, chunk), pl.ds(0, D_MODEL)]
            return pltpu.make_async_copy(rows_v.at[b], dst, sem)

        for b in range(LOOKAHEAD):
            gather(b, b, gsems[b]).start()

        def outer_body(t, _):
            for b in range(NBUF):
                i = t * NBUF + b
                j = i + LOOKAHEAD
                bj = (b + LOOKAHEAD) % NBUF

                @pl.when(j < num_chunks)
                def _():
                    @pl.when(j >= NBUF)
                    def _():
                        scatter(j - NBUF, bj, ssems[bj]).wait()

                    gather(j, bj, gsems[bj]).start()

                gather(i, b, gsems[b]).wait()
                scatter(i, b, ssems[b]).start()
            return 0

        lax.fori_loop(0, outer, outer_body, 0)
        for b in range(NBUF):
            scatter(num_chunks - NBUF + b, b, ssems[b]).wait()

    return k(table_p, x_flat)


def kernel(x, table):
    total_rows = x.shape[0] * x.shape[1]
    x_flat = x.reshape(total_rows).astype(jnp.int32)
    main = lax.slice(table, (0, 0), (MAIN_ROWS, D_MODEL))
    t4 = jnp.transpose(
        jnp.reshape(jnp.transpose(main), (8, 8, MAIN_TILES, D_PAD)), (0, 2, 1, 3)
    )
    tail = lax.slice(table, (MAIN_ROWS, 0), (VOCAB, D_MODEL))
    table_p = _transpose_scale(t4, tail)
    out = _embed(x_flat, table_p, total_rows=total_rows, chunk=400)
    out = out.reshape(x.shape[0], x.shape[1], D_PAD)[:, :, :D_MODEL]
    return out


# speedup vs baseline: 1.4920x; 1.4920x over previous
import functools
import math

import jax
import jax.numpy as jnp
from jax import lax
from jax.experimental import pallas as pl
from jax.experimental.pallas import tpu as pltpu
from jax.experimental.pallas import tpu_sc as plsc

D_MODEL = 64
D_PAD = 128
SCALE = math.sqrt(D_MODEL)

NUM_CORES = 2
NUM_SUBCORES = 16
NUM_WORKERS = NUM_CORES * NUM_SUBCORES
LANES = 16
NBUF = 4
LOOKAHEAD = 2


@functools.partial(jax.jit, static_argnames=("total_rows", "chunk"))
def _embed(x_flat, table_c, *, total_rows, chunk):
    rows_per_worker = total_rows // NUM_WORKERS
    num_chunks = rows_per_worker // chunk
    assert num_chunks % NBUF == 0
    outer = num_chunks // NBUF
    mesh = plsc.VectorSubcoreMesh(core_axis_name="c", subcore_axis_name="s")

    @functools.partial(
        pl.kernel,
        mesh=mesh,
        out_type=jax.ShapeDtypeStruct((total_rows, D_PAD), jnp.float32),
        scratch_types=[
            pltpu.VMEM((rows_per_worker,), jnp.int32),
            pltpu.VMEM((NBUF, chunk, D_MODEL), jnp.float32),
            [pltpu.SemaphoreType.DMA] * NBUF,
            [pltpu.SemaphoreType.DMA] * NBUF,
        ],
        compiler_params=pltpu.CompilerParams(use_tc_tiling_on_sc=False),
    )
    def k(table_hbm, idx_hbm, out_hbm, idx_v, rows_v, gsems, ssems):
        wid = lax.axis_index("s") * NUM_CORES + lax.axis_index("c")
        base = wid * rows_per_worker
        pltpu.sync_copy(idx_hbm.at[pl.ds(base, rows_per_worker)], idx_v)

        def gather(i, b, sem):
            idx_sl = idx_v.at[pl.ds(i * chunk, chunk)]
            return pltpu.make_async_copy(table_hbm.at[idx_sl], rows_v.at[b], sem)

        def scatter(i, b, sem):
            dst = out_hbm.at[pl.ds(base + i * chunk, chunk), pl.ds(0, D_MODEL)]
            return pltpu.make_async_copy(rows_v.at[b], dst, sem)

        for b in range(LOOKAHEAD):
            gather(b, b, gsems[b]).start()

        def outer_body(t, _):
            for b in range(NBUF):
                i = t * NBUF + b
                j = i + LOOKAHEAD
                bj = (b + LOOKAHEAD) % NBUF

                @pl.when(j < num_chunks)
                def _():
                    @pl.when(j >= NBUF)
                    def _():
                        scatter(j - NBUF, bj, ssems[bj]).wait()

                    gather(j, bj, gsems[bj]).start()

                gather(i, b, gsems[b]).wait()

                @plsc.parallel_loop(0, chunk, unroll=4)
                def _(r):
                    for jj in range(D_MODEL // LANES):
                        sl = pl.ds(jj * LANES, LANES)
                        rows_v[b, r, sl] = rows_v[b, r, sl] * SCALE

                scatter(i, b, ssems[b]).start()
            return 0

        lax.fori_loop(0, outer, outer_body, 0)
        for b in range(NBUF):
            scatter(num_chunks - NBUF + b, b, ssems[b]).wait()

    return k(table_c, x_flat)


def kernel(x, table):
    total_rows = x.shape[0] * x.shape[1]
    x_flat = x.reshape(total_rows).astype(jnp.int32)
    out = _embed(x_flat, table, total_rows=total_rows, chunk=400)
    out = out.reshape(x.shape[0], x.shape[1], D_PAD)[:, :, :D_MODEL]
    return out


# speedup vs baseline: 1.6041x; 1.0751x over previous
import functools
import math

import jax
import jax.numpy as jnp
from jax import lax
from jax.experimental import pallas as pl
from jax.experimental.pallas import tpu as pltpu
from jax.experimental.pallas import tpu_sc as plsc

D_MODEL = 64
D_PAD = 128
SCALE = math.sqrt(D_MODEL)

NUM_CORES = 2
NUM_SUBCORES = 16
NUM_WORKERS = NUM_CORES * NUM_SUBCORES
LANES = 16
NBUF = 4
LOOKAHEAD = 2


@functools.partial(jax.jit, static_argnames=("total_rows", "chunk"))
def _embed(x_flat, table_c, *, total_rows, chunk):
    rows_per_worker = total_rows // NUM_WORKERS
    num_chunks = rows_per_worker // chunk
    assert num_chunks % NBUF == 0
    outer = num_chunks // NBUF
    mesh = plsc.VectorSubcoreMesh(core_axis_name="c", subcore_axis_name="s")

    @functools.partial(
        pl.kernel,
        mesh=mesh,
        out_type=jax.ShapeDtypeStruct((total_rows, D_PAD), jnp.float32),
        scratch_types=[
            pltpu.VMEM((rows_per_worker,), jnp.int32),
            pltpu.VMEM((NBUF, chunk, D_MODEL), jnp.float32),
            [pltpu.SemaphoreType.DMA] * NBUF,
            [pltpu.SemaphoreType.DMA] * NBUF,
        ],
        compiler_params=pltpu.CompilerParams(use_tc_tiling_on_sc=False),
    )
    def k(table_hbm, idx_hbm, out_hbm, idx_v, rows_v, gsems, ssems):
        wid = lax.axis_index("s") * NUM_CORES + lax.axis_index("c")
        base = wid * rows_per_worker
        pltpu.sync_copy(idx_hbm.at[pl.ds(base, rows_per_worker)], idx_v)

        def gather(i, b, sem):
            idx_sl = idx_v.at[pl.ds(i * chunk, chunk)]
            return pltpu.make_async_copy(table_hbm.at[idx_sl], rows_v.at[b], sem)

        def scatter(i, b, sem):
            dst = out_hbm.at[pl.ds(base + i * chunk, chunk), pl.ds(0, D_MODEL)]
            return pltpu.make_async_copy(rows_v.at[b], dst, sem)

        for b in range(LOOKAHEAD):
            gather(b, b, gsems[b]).start()

        def outer_body(t, _):
            for b in range(NBUF):
                i = t * NBUF + b
                j = i + LOOKAHEAD
                bj = (b + LOOKAHEAD) % NBUF

                @pl.when(j < num_chunks)
                def _():
                    @pl.when(j >= NBUF)
                    def _():
                        scatter(j - NBUF, bj, ssems[bj]).wait()

                    gather(j, bj, gsems[bj]).start()

                gather(i, b, gsems[b]).wait()

                @plsc.parallel_loop(0, chunk, unroll=4)
                def _(r):
                    for jj in range(D_MODEL // LANES):
                        sl = pl.ds(jj * LANES, LANES)
                        rows_v[b, r, sl] = rows_v[b, r, sl] * SCALE

                scatter(i, b, ssems[b]).start()
            return 0

        lax.fori_loop(0, outer, outer_body, 0)
        for b in range(NBUF):
            scatter(num_chunks - NBUF + b, b, ssems[b]).wait()

    return k(table_c, x_flat)


def kernel(x, table):
    total_rows = x.shape[0] * x.shape[1]
    x_flat = x.reshape(total_rows).astype(jnp.int32) * 2
    vocab = table.shape[0]
    table_p = jnp.pad(table, ((0, 0), (0, D_PAD - D_MODEL)))
    table_v = table_p.reshape(2 * vocab, D_MODEL)
    out = _embed(x_flat, table_v, total_rows=total_rows, chunk=400)
    out = out.reshape(x.shape[0], x.shape[1], D_PAD)[:, :, :D_MODEL]
    return out
